# initial kernel scaffold (unmeasured)
import jax
import jax.numpy as jnp
from jax import lax
from jax.experimental import pallas as pl
from jax.experimental.pallas import tpu as pltpu

B, SQ, H, D = 8, 8, 16, 128
SKV = 1024
SCALE = D ** -0.5


def _flash_body(q_ref, k_ref, v_ref, o_ref, m_ref, l_ref):
    q = q_ref[0, :, 0, :]
    k = k_ref[0, :, 0, :]
    v = v_ref[0, :, 0, :]
    s = lax.dot_general(q, k, (((1,), (1,)), ((), ())),
                        preferred_element_type=jnp.float32) * SCALE
    m = jnp.max(s, axis=-1, keepdims=True)
    p = jnp.exp(s - m)
    l = jnp.sum(p, axis=-1, keepdims=True)
    o = jnp.dot(p, v, preferred_element_type=jnp.float32)
    o_ref[...] = o.reshape(1, SQ, 1, D)
    m_ref[...] = m.reshape(1, SQ, 1)
    l_ref[...] = l.reshape(1, SQ, 1)


def _combine_body(o_ref, m_ref, l_ref, out_ref, o_rx, m_rx, l_rx,
                  send_sems, recv_sems):
    my_x = lax.axis_index("x")
    my_y = lax.axis_index("y")
    peer = (my_x, 1 - my_y)
    copies = []
    for i, (src, dst) in enumerate(
        [(o_ref, o_rx), (m_ref, m_rx), (l_ref, l_rx)]
    ):
        c = pltpu.make_async_remote_copy(
            src_ref=src, dst_ref=dst,
            send_sem=send_sems.at[i], recv_sem=recv_sems.at[i],
            device_id=peer, device_id_type=pl.DeviceIdType.MESH)
        c.start()
        copies.append(c)
    for c in copies:
        c.wait()
    m_a = m_ref[...]
    m_b = m_rx[...]
    m_n = jnp.maximum(m_a, m_b)
    ea = jnp.exp(m_a - m_n)
    eb = jnp.exp(m_b - m_n)
    l_n = l_ref[...] * ea + l_rx[...] * eb
    o_n = o_ref[...] * ea[..., None] + o_rx[...] * eb[..., None]
    out_ref[...] = o_n / l_n[..., None]


def kernel(Q, K, V):
    o_p, m_p, l_p = pl.pallas_call(
        _flash_body,
        grid=(B, H),
        in_specs=[
            pl.BlockSpec((1, SQ, 1, D), lambda b, h: (b, 0, h, 0)),
            pl.BlockSpec((1, SKV, 1, D), lambda b, h: (b, 0, h, 0)),
            pl.BlockSpec((1, SKV, 1, D), lambda b, h: (b, 0, h, 0)),
        ],
        out_specs=[
            pl.BlockSpec((1, SQ, 1, D), lambda b, h: (b, 0, h, 0)),
            pl.BlockSpec((1, SQ, 1), lambda b, h: (b, 0, h)),
            pl.BlockSpec((1, SQ, 1), lambda b, h: (b, 0, h)),
        ],
        out_shape=[
            jax.ShapeDtypeStruct((B, SQ, H, D), jnp.float32),
            jax.ShapeDtypeStruct((B, SQ, H), jnp.float32),
            jax.ShapeDtypeStruct((B, SQ, H), jnp.float32),
        ],
    )(Q, K, V)

    out = pl.pallas_call(
        _combine_body,
        in_specs=[pl.BlockSpec(memory_space=pltpu.VMEM)] * 3,
        out_specs=pl.BlockSpec(memory_space=pltpu.VMEM),
        out_shape=jax.ShapeDtypeStruct((B, SQ, H, D), jnp.float32),
        scratch_shapes=[
            pltpu.VMEM((B, SQ, H, D), jnp.float32),
            pltpu.VMEM((B, SQ, H), jnp.float32),
            pltpu.VMEM((B, SQ, H), jnp.float32),
            pltpu.SemaphoreType.DMA((3,)),
            pltpu.SemaphoreType.DMA((3,)),
        ],
        compiler_params=pltpu.CompilerParams(collective_id=0),
    )(o_p, m_p, l_p)
    return out


# baseline (device time: 138040 ns/iter reference)
import jax
import jax.numpy as jnp
from jax import lax
from jax.experimental import pallas as pl
from jax.experimental.pallas import tpu as pltpu

B, SQ, H, D = 8, 8, 16, 128
SKV = 1024
CHUNK = SKV
SCALE = D ** -0.5
NSTEP = B * H


def _flash_body(q_hbm, k_hbm, v_hbm, o_ref, m_ref, l_ref,
                qbuf, kbuf, vbuf, qsem, ksem, vsem):
    b = pl.program_id(0)
    h = pl.program_id(1)
    i = b * H + h

    def dmas(idx, slot):
        bb = idx // H
        hh = lax.rem(idx, H)
        return (
            pltpu.make_async_copy(q_hbm.at[bb, :, hh, :], qbuf.at[slot],
                                  qsem.at[slot]),
            pltpu.make_async_copy(k_hbm.at[bb, :, hh, :], kbuf.at[slot],
                                  ksem.at[slot]),
            pltpu.make_async_copy(v_hbm.at[bb, :, hh, :], vbuf.at[slot],
                                  vsem.at[slot]),
        )

    slot = lax.rem(i, 2)
    nslot = lax.rem(i + 1, 2)

    @pl.when(i == 0)
    def _():
        for c in dmas(i, slot):
            c.start()

    @pl.when(i + 1 < NSTEP)
    def _():
        for c in dmas(i + 1, nslot):
            c.start()

    for c in dmas(i, slot):
        c.wait()

    q = qbuf[slot]
    k = kbuf[slot]
    v = vbuf[slot]
    s = lax.dot_general(q, k, (((1,), (1,)), ((), ())),
                        preferred_element_type=jnp.float32) * SCALE
    m = jnp.max(s, axis=-1, keepdims=True)
    p = jnp.exp(s - m)
    l = jnp.sum(p, axis=-1, keepdims=True)
    o = jnp.dot(p, v, preferred_element_type=jnp.float32)
    o_ref[...] = o.reshape(1, 1, SQ, D)
    m_ref[...] = m.reshape(1, 1, SQ, 1)
    l_ref[...] = l.reshape(1, 1, SQ, 1)


def _combine_body(o_ref, m_ref, l_ref, out_ref, o_rx, m_rx, l_rx,
                  send_sems, recv_sems):
    my_x = lax.axis_index("x")
    my_y = lax.axis_index("y")
    peer = (my_x, 1 - my_y)
    copies = []
    for i, (src, dst) in enumerate(
        [(o_ref, o_rx), (m_ref, m_rx), (l_ref, l_rx)]
    ):
        c = pltpu.make_async_remote_copy(
            src_ref=src, dst_ref=dst,
            send_sem=send_sems.at[i], recv_sem=recv_sems.at[i],
            device_id=peer, device_id_type=pl.DeviceIdType.MESH)
        c.start()
        copies.append(c)
    for c in copies:
        c.wait()
    m_a = m_ref[...]
    m_b = m_rx[...]
    m_n = jnp.maximum(m_a, m_b)
    ea = jnp.exp(m_a - m_n)
    eb = jnp.exp(m_b - m_n)
    l_n = l_ref[...] * ea + l_rx[...] * eb
    o_n = o_ref[...] * ea + o_rx[...] * eb
    o_n = o_n / l_n
    out_ref[...] = jnp.transpose(o_n, (0, 2, 1, 3))


def kernel(Q, K, V):
    o_p, m_p, l_p = pl.pallas_call(
        _flash_body,
        grid=(B, H),
        in_specs=[
            pl.BlockSpec(memory_space=pl.ANY),
            pl.BlockSpec(memory_space=pl.ANY),
            pl.BlockSpec(memory_space=pl.ANY),
        ],
        out_specs=[
            pl.BlockSpec((1, 1, SQ, D), lambda b, h: (b, h, 0, 0)),
            pl.BlockSpec((1, 1, SQ, 1), lambda b, h: (b, h, 0, 0)),
            pl.BlockSpec((1, 1, SQ, 1), lambda b, h: (b, h, 0, 0)),
        ],
        out_shape=[
            jax.ShapeDtypeStruct((B, H, SQ, D), jnp.float32),
            jax.ShapeDtypeStruct((B, H, SQ, 1), jnp.float32),
            jax.ShapeDtypeStruct((B, H, SQ, 1), jnp.float32),
        ],
        scratch_shapes=[
            pltpu.VMEM((2, SQ, D), jnp.float32),
            pltpu.VMEM((2, CHUNK, D), jnp.float32),
            pltpu.VMEM((2, CHUNK, D), jnp.float32),
            pltpu.SemaphoreType.DMA((2,)),
            pltpu.SemaphoreType.DMA((2,)),
            pltpu.SemaphoreType.DMA((2,)),
        ],
    )(Q, K, V)

    out = pl.pallas_call(
        _combine_body,
        in_specs=[pl.BlockSpec(memory_space=pltpu.VMEM)] * 3,
        out_specs=pl.BlockSpec(memory_space=pltpu.VMEM),
        out_shape=jax.ShapeDtypeStruct((B, SQ, H, D), jnp.float32),
        scratch_shapes=[
            pltpu.VMEM((B, H, SQ, D), jnp.float32),
            pltpu.VMEM((B, H, SQ, 1), jnp.float32),
            pltpu.VMEM((B, H, SQ, 1), jnp.float32),
            pltpu.SemaphoreType.DMA((3,)),
            pltpu.SemaphoreType.DMA((3,)),
        ],
    )(o_p, m_p, l_p)
    return out


# device time: 92474 ns/iter; 1.4927x vs baseline; 1.4927x over previous
import jax
import jax.numpy as jnp
from jax import lax
from jax.experimental import pallas as pl
from jax.experimental.pallas import tpu as pltpu

B, SQ, H, D = 8, 8, 16, 128
SKV = 1024
CHUNK = SKV // 2
SCALE = D ** -0.5


def _flash_body(q_ref, k_hbm, v_hbm, o_ref, m_ref, l_ref,
                kbuf, vbuf, ksem, vsem):
    b = pl.program_id(0)
    my_x = lax.axis_index("x")
    row0 = my_x * CHUNK

    def dmas(bb, slot):
        return (
            pltpu.make_async_copy(k_hbm.at[bb, pl.ds(row0, CHUNK), :, :],
                                  kbuf.at[slot], ksem.at[slot]),
            pltpu.make_async_copy(v_hbm.at[bb, pl.ds(row0, CHUNK), :, :],
                                  vbuf.at[slot], vsem.at[slot]),
        )

    slot = lax.rem(b, 2)
    nslot = lax.rem(b + 1, 2)

    @pl.when(b == 0)
    def _():
        for c in dmas(b, slot):
            c.start()

    @pl.when(b + 1 < B)
    def _():
        for c in dmas(b + 1, nslot):
            c.start()

    for c in dmas(b, slot):
        c.wait()

    for h in range(H):
        q = q_ref[b, :, h, :]
        k = kbuf[slot, :, h, :]
        v = vbuf[slot, :, h, :]
        s = lax.dot_general(q, k, (((1,), (1,)), ((), ())),
                            preferred_element_type=jnp.float32) * SCALE
        m = jnp.max(s, axis=-1, keepdims=True)
        p = jnp.exp(s - m)
        l = jnp.sum(p, axis=-1, keepdims=True)
        o = jnp.dot(p, v, preferred_element_type=jnp.float32)
        o_ref[0, h] = o
        m_ref[0, h] = m
        l_ref[0, h] = l


def _combine(m_a, l_a, o_a, m_b, l_b, o_b):
    m_n = jnp.maximum(m_a, m_b)
    ea = jnp.exp(m_a - m_n)
    eb = jnp.exp(m_b - m_n)
    return m_n, l_a * ea + l_b * eb, o_a * ea + o_b * eb


def _combine_body(o_ref, m_ref, l_ref, out_ref,
                  oc, mc, lc, o_rx, m_rx, l_rx, o_ry, m_ry, l_ry,
                  send_sems, recv_sems):
    my_x = lax.axis_index("x")
    my_y = lax.axis_index("y")

    xcopies = []
    for i, (src, dst) in enumerate(
        [(o_ref, o_rx), (m_ref, m_rx), (l_ref, l_rx)]
    ):
        c = pltpu.make_async_remote_copy(
            src_ref=src, dst_ref=dst,
            send_sem=send_sems.at[i], recv_sem=recv_sems.at[i],
            device_id=(1 - my_x, my_y), device_id_type=pl.DeviceIdType.MESH)
        c.start()
        xcopies.append(c)
    for c in xcopies:
        c.wait()

    m_n, l_n, o_n = _combine(m_ref[...], l_ref[...], o_ref[...],
                             m_rx[...], l_rx[...], o_rx[...])
    mc[...] = m_n
    lc[...] = l_n
    oc[...] = o_n

    ycopies = []
    for i, (src, dst) in enumerate(
        [(oc, o_ry), (mc, m_ry), (lc, l_ry)]
    ):
        c = pltpu.make_async_remote_copy(
            src_ref=src, dst_ref=dst,
            send_sem=send_sems.at[3 + i], recv_sem=recv_sems.at[3 + i],
            device_id=(my_x, 1 - my_y), device_id_type=pl.DeviceIdType.MESH)
        c.start()
        ycopies.append(c)
    for c in ycopies:
        c.wait()

    m_f, l_f, o_f = _combine(mc[...], lc[...], oc[...],
                             m_ry[...], l_ry[...], o_ry[...])
    o_f = o_f / l_f
    out_ref[...] = jnp.transpose(o_f, (0, 2, 1, 3))


def kernel(Q, K, V):
    o_p, m_p, l_p = pl.pallas_call(
        _flash_body,
        grid=(B,),
        in_specs=[
            pl.BlockSpec(memory_space=pltpu.VMEM),
            pl.BlockSpec(memory_space=pl.ANY),
            pl.BlockSpec(memory_space=pl.ANY),
        ],
        out_specs=[
            pl.BlockSpec((1, H, SQ, D), lambda b: (b, 0, 0, 0)),
            pl.BlockSpec((1, H, SQ, 1), lambda b: (b, 0, 0, 0)),
            pl.BlockSpec((1, H, SQ, 1), lambda b: (b, 0, 0, 0)),
        ],
        out_shape=[
            jax.ShapeDtypeStruct((B, H, SQ, D), jnp.float32),
            jax.ShapeDtypeStruct((B, H, SQ, 1), jnp.float32),
            jax.ShapeDtypeStruct((B, H, SQ, 1), jnp.float32),
        ],
        scratch_shapes=[
            pltpu.VMEM((2, CHUNK, H, D), jnp.float32),
            pltpu.VMEM((2, CHUNK, H, D), jnp.float32),
            pltpu.SemaphoreType.DMA((2,)),
            pltpu.SemaphoreType.DMA((2,)),
        ],
    )(Q, K, V)

    out = pl.pallas_call(
        _combine_body,
        in_specs=[pl.BlockSpec(memory_space=pltpu.VMEM)] * 3,
        out_specs=pl.BlockSpec(memory_space=pltpu.VMEM),
        out_shape=jax.ShapeDtypeStruct((B, SQ, H, D), jnp.float32),
        scratch_shapes=[
            pltpu.VMEM((B, H, SQ, D), jnp.float32),
            pltpu.VMEM((B, H, SQ, 1), jnp.float32),
            pltpu.VMEM((B, H, SQ, 1), jnp.float32),
            pltpu.VMEM((B, H, SQ, D), jnp.float32),
            pltpu.VMEM((B, H, SQ, 1), jnp.float32),
            pltpu.VMEM((B, H, SQ, 1), jnp.float32),
            pltpu.VMEM((B, H, SQ, D), jnp.float32),
            pltpu.VMEM((B, H, SQ, 1), jnp.float32),
            pltpu.VMEM((B, H, SQ, 1), jnp.float32),
            pltpu.SemaphoreType.DMA((6,)),
            pltpu.SemaphoreType.DMA((6,)),
        ],
    )(o_p, m_p, l_p)
    return out
